# Initial kernel scaffold; baseline (speedup 1.0000x reference)
import jax
import jax.numpy as jnp
from jax.experimental import pallas as pl


def kernel(x, edge_index, W, gamma, beta):
    raise NotImplementedError("write your pallas kernel here")


# speedup vs baseline: 20.8566x; 20.8566x over previous
import jax
import jax.numpy as jnp
from jax import lax
from jax.experimental import pallas as pl
from jax.experimental.pallas import tpu as pltpu
from jax.experimental.pallas import tpu_sc as plsc

N = 10000
E = 320000
D = 128
EPS = 1e-5

NC = 2
NS = 16
NW = NC * NS
EPW = E // NW
NPW = 624
NTAIL = N - NS * NPW
K = 80
NCHUNK = EPW // K

_mesh = lambda: plsc.VectorSubcoreMesh(core_axis_name="c", subcore_axis_name="s")


def _deg_body(src_hbm, zeros_hbm, out_hbm, idx_all, hist_v):
    cid = lax.axis_index("c")
    sid = lax.axis_index("s")
    wid = sid * NC + cid
    pltpu.sync_copy(zeros_hbm, hist_v)
    pltpu.sync_copy(src_hbm.at[pl.ds(wid * EPW, EPW)], idx_all)
    ones = jnp.ones((16,), jnp.float32)

    def step(k, carry):
        idx16 = idx_all[pl.ds(k * 16, 16)]
        plsc.addupdate_scatter(hist_v, [idx16], ones)
        return carry

    lax.fori_loop(0, EPW // 16, step, 0)
    pltpu.sync_copy(hist_v, out_hbm.at[pl.ds(wid * N, N)])


def _deg_call(src, zeros1):
    f = pl.kernel(
        _deg_body,
        out_type=jax.ShapeDtypeStruct((NW * N,), jnp.float32),
        mesh=_mesh(),
        compiler_params=pltpu.CompilerParams(needs_layout_passes=False),
        scratch_types=[
            pltpu.VMEM((EPW,), jnp.int32),
            pltpu.VMEM((N,), jnp.float32),
        ],
    )
    return f(src, zeros1)


def _agg_body(hs_hbm, src_hbm, dst_hbm, zeros_hbm, out_hbm,
              idxs_v, idxd_v, rows_v, acc_sp, sem):
    cid = lax.axis_index("c")
    sid = lax.axis_index("s")
    wid = sid * NC + cid
    pltpu.sync_copy(zeros_hbm.at[pl.ds(sid * NPW, NPW)],
                    acc_sp.at[pl.ds(sid * NPW, NPW)])

    @pl.when(sid == 0)
    def _():
        pltpu.sync_copy(zeros_hbm.at[pl.ds(NS * NPW, NTAIL)],
                        acc_sp.at[pl.ds(NS * NPW, NTAIL)])

    plsc.subcore_barrier()
    base = wid * EPW

    def step(j, carry):
        off = base + j * K
        pltpu.sync_copy(src_hbm.at[pl.ds(off, K)], idxs_v)
        pltpu.sync_copy(dst_hbm.at[pl.ds(off, K)], idxd_v)
        pltpu.async_copy(hs_hbm.at[idxs_v], rows_v, sem).wait()
        pltpu.sync_copy(rows_v, acc_sp.at[idxd_v], add=True)
        return carry

    lax.fori_loop(0, NCHUNK, step, 0)
    plsc.subcore_barrier()
    pltpu.sync_copy(acc_sp.at[pl.ds(sid * NPW, NPW)],
                    out_hbm.at[pl.ds(cid * N + sid * NPW, NPW)])

    @pl.when(sid == 0)
    def _():
        pltpu.sync_copy(acc_sp.at[pl.ds(NS * NPW, NTAIL)],
                        out_hbm.at[pl.ds(cid * N + NS * NPW, NTAIL)])


def _agg_call(hs, src, dst, zeros2):
    f = pl.kernel(
        _agg_body,
        out_type=jax.ShapeDtypeStruct((NC * N, D), jnp.float32),
        mesh=_mesh(),
        scratch_types=[
            pltpu.VMEM((K,), jnp.int32),
            pltpu.VMEM((K,), jnp.int32),
            pltpu.VMEM((K, D), jnp.float32),
            pltpu.VMEM_SHARED((N, D), jnp.float32),
            pltpu.SemaphoreType.DMA,
        ],
    )
    return f(hs, src, dst, zeros2)


def _mm_body(cnt_ref, x_ref, w_ref, hs_ref):
    deg = jnp.sum(cnt_ref[...], axis=0) + 1.0
    dis = lax.rsqrt(deg)
    xs = x_ref[...] * dis[:, None]
    hs_ref[...] = lax.dot_general(xs, w_ref[...], (((1,), (1,)), ((), ())),
                                  preferred_element_type=jnp.float32)


def _mm_call(cnt, x, W):
    return pl.pallas_call(
        _mm_body,
        out_shape=jax.ShapeDtypeStruct((N, D), jnp.float32),
    )(cnt, x, W)


def _bn_body(cnt_ref, accp_ref, hs_ref, g_ref, b_ref, o_ref):
    deg = jnp.sum(cnt_ref[...], axis=0) + 1.0
    dis = lax.rsqrt(deg)
    a = accp_ref[0:N, :] + accp_ref[N:2 * N, :] + hs_ref[...]
    z = jnp.maximum(a * dis[:, None], 0.0)
    mean = jnp.mean(z, axis=0, keepdims=True)
    var = jnp.mean(z * z, axis=0, keepdims=True) - mean * mean
    scale = g_ref[...] * lax.rsqrt(var + EPS)
    o_ref[...] = (z - mean) * scale + b_ref[...]


def _bn_call(cnt, accp, hs, gamma, beta):
    return pl.pallas_call(
        _bn_body,
        out_shape=jax.ShapeDtypeStruct((N, D), jnp.float32),
    )(cnt, accp, hs, gamma.reshape(1, D), beta.reshape(1, D))


def kernel(x, edge_index, W, gamma, beta):
    src = edge_index[0]
    dst = edge_index[1]
    zeros1 = jnp.zeros((N,), jnp.float32)
    zeros2 = jnp.zeros((N, D), jnp.float32)

    cnt = _deg_call(src, zeros1).reshape(NW, N)
    hs = _mm_call(cnt, x, W)
    accp = _agg_call(hs, src, dst, zeros2)
    return _bn_call(cnt, accp, hs, gamma, beta)


# speedup vs baseline: 32.5011x; 1.5583x over previous
import jax
import jax.numpy as jnp
from jax import lax
from jax.experimental import pallas as pl
from jax.experimental.pallas import tpu as pltpu
from jax.experimental.pallas import tpu_sc as plsc

N = 10000
E = 320000
D = 128
EPS = 1e-5

NC = 2
NS = 16
NW = NC * NS
EPW = E // NW
NPW = 624
NTAIL = N - NS * NPW
K = 64
NFULL = EPW // K
KTAIL = EPW - NFULL * K
NPAIR = NFULL // 2

_mesh = lambda: plsc.VectorSubcoreMesh(core_axis_name="c", subcore_axis_name="s")


def _deg_body(src_hbm, zeros_hbm, out_hbm, idx_all, hist_v):
    cid = lax.axis_index("c")
    sid = lax.axis_index("s")
    wid = sid * NC + cid
    pltpu.sync_copy(zeros_hbm, hist_v)
    pltpu.sync_copy(src_hbm.at[pl.ds(wid * EPW, EPW)], idx_all)
    ones = jnp.ones((16,), jnp.float32)

    def step(k, carry):
        idx16 = idx_all[pl.ds(k * 16, 16)]
        plsc.addupdate_scatter(hist_v, [idx16], ones)
        return carry

    lax.fori_loop(0, EPW // 16, step, 0)
    pltpu.sync_copy(hist_v, out_hbm.at[pl.ds(wid * N, N)])


def _deg_call(src, zeros1):
    f = pl.kernel(
        _deg_body,
        out_type=jax.ShapeDtypeStruct((NW * N,), jnp.float32),
        mesh=_mesh(),
        compiler_params=pltpu.CompilerParams(needs_layout_passes=False),
        scratch_types=[
            pltpu.VMEM((EPW,), jnp.int32),
            pltpu.VMEM((N,), jnp.float32),
        ],
    )
    return f(src, zeros1)


def _agg_body(hs_hbm, src_hbm, dst_hbm, zeros_hbm, out_hbm,
              src_all, dst_all, idxd0, idxd1, rows0, rows1,
              idxd_t, rows_t, acc_sp, sem):
    cid = lax.axis_index("c")
    sid = lax.axis_index("s")
    wid = sid * NC + cid
    pltpu.sync_copy(zeros_hbm.at[pl.ds(sid * NPW, NPW)],
                    acc_sp.at[pl.ds(sid * NPW, NPW)])

    @pl.when(sid == 0)
    def _():
        pltpu.sync_copy(zeros_hbm.at[pl.ds(NS * NPW, NTAIL)],
                        acc_sp.at[pl.ds(NS * NPW, NTAIL)])

    base = wid * EPW
    pltpu.sync_copy(src_hbm.at[pl.ds(base, EPW)], src_all)
    pltpu.sync_copy(dst_hbm.at[pl.ds(base, EPW)], dst_all)
    plsc.subcore_barrier()

    def copy_idx(off, buf, n=K):
        for t in range(n // 16):
            buf[pl.ds(t * 16, 16)] = dst_all[pl.ds(off + t * 16, 16)]

    def gather_start(c, rows):
        return pltpu.async_copy(hs_hbm.at[src_all.at[pl.ds(c * K, K)]],
                                rows, sem)

    def gather_wait(c, rows):
        pltpu.make_async_copy(hs_hbm.at[src_all.at[pl.ds(c * K, K)]],
                              rows, sem).wait()

    copy_idx(0, idxd0)
    gather_start(0, rows0)

    def step(j, carry):
        c0 = 2 * j
        c1 = c0 + 1
        copy_idx(c1 * K, idxd1)
        gather_wait(c0, rows0)
        gather_start(c1, rows1)
        pltpu.sync_copy(rows0, acc_sp.at[idxd0], add=True)

        @pl.when(j < NPAIR - 1)
        def _():
            copy_idx((c0 + 2) * K, idxd0)

        gather_wait(c1, rows1)

        @pl.when(j < NPAIR - 1)
        def _():
            gather_start(c0 + 2, rows0)

        pltpu.sync_copy(rows1, acc_sp.at[idxd1], add=True)
        return carry

    lax.fori_loop(0, NPAIR, step, 0)
    copy_idx(NFULL * K, idxd_t, n=KTAIL)
    pltpu.async_copy(hs_hbm.at[src_all.at[pl.ds(NFULL * K, KTAIL)]],
                     rows_t, sem).wait()
    pltpu.sync_copy(rows_t, acc_sp.at[idxd_t], add=True)
    plsc.subcore_barrier()
    pltpu.sync_copy(acc_sp.at[pl.ds(sid * NPW, NPW)],
                    out_hbm.at[pl.ds(cid * N + sid * NPW, NPW)])

    @pl.when(sid == 0)
    def _():
        pltpu.sync_copy(acc_sp.at[pl.ds(NS * NPW, NTAIL)],
                        out_hbm.at[pl.ds(cid * N + NS * NPW, NTAIL)])


def _agg_call(hs, src, dst, zeros2):
    f = pl.kernel(
        _agg_body,
        out_type=jax.ShapeDtypeStruct((NC * N, D), jnp.float32),
        mesh=_mesh(),
        scratch_types=[
            pltpu.VMEM((EPW,), jnp.int32),
            pltpu.VMEM((EPW,), jnp.int32),
            pltpu.VMEM((K,), jnp.int32),
            pltpu.VMEM((K,), jnp.int32),
            pltpu.VMEM((K, D), jnp.float32),
            pltpu.VMEM((K, D), jnp.float32),
            pltpu.VMEM((KTAIL,), jnp.int32),
            pltpu.VMEM((KTAIL, D), jnp.float32),
            pltpu.VMEM_SHARED((N, D), jnp.float32),
            pltpu.SemaphoreType.DMA,
        ],
    )
    return f(hs, src, dst, zeros2)


def _mm_body(cnt_ref, x_ref, w_ref, hs_ref):
    deg = jnp.sum(cnt_ref[...], axis=0) + 1.0
    dis = lax.rsqrt(deg)
    xs = x_ref[...] * dis[:, None]
    hs_ref[...] = lax.dot_general(xs, w_ref[...], (((1,), (1,)), ((), ())),
                                  preferred_element_type=jnp.float32)


def _mm_call(cnt, x, W):
    return pl.pallas_call(
        _mm_body,
        out_shape=jax.ShapeDtypeStruct((N, D), jnp.float32),
    )(cnt, x, W)


def _bn_body(cnt_ref, accp_ref, hs_ref, g_ref, b_ref, o_ref):
    deg = jnp.sum(cnt_ref[...], axis=0) + 1.0
    dis = lax.rsqrt(deg)
    a = accp_ref[0:N, :] + accp_ref[N:2 * N, :] + hs_ref[...]
    z = jnp.maximum(a * dis[:, None], 0.0)
    mean = jnp.mean(z, axis=0, keepdims=True)
    var = jnp.mean(z * z, axis=0, keepdims=True) - mean * mean
    scale = g_ref[...] * lax.rsqrt(var + EPS)
    o_ref[...] = (z - mean) * scale + b_ref[...]


def _bn_call(cnt, accp, hs, gamma, beta):
    return pl.pallas_call(
        _bn_body,
        out_shape=jax.ShapeDtypeStruct((N, D), jnp.float32),
    )(cnt, accp, hs, gamma.reshape(1, D), beta.reshape(1, D))


def kernel(x, edge_index, W, gamma, beta):
    src = edge_index[0]
    dst = edge_index[1]
    zeros1 = jnp.zeros((N,), jnp.float32)
    zeros2 = jnp.zeros((N, D), jnp.float32)

    cnt = _deg_call(src, zeros1).reshape(NW, N)
    hs = _mm_call(cnt, x, W)
    accp = _agg_call(hs, src, dst, zeros2)
    return _bn_call(cnt, accp, hs, gamma, beta)
